# baseline (speedup 1.0000x reference)
import functools

import jax
import jax.numpy as jnp
from jax import lax
from jax.experimental import pallas as pl
from jax.experimental.pallas import tpu as pltpu
from jax.experimental.pallas import tpu_sc as plsc

_NC = 2
_NS = 16
_NW = _NC * _NS
_K = 128
_D = 128


def _SPLIT0(total2):
    c0 = int(round(total2 * 0.90 / 2)) * 2
    return min(max(c0, 2), total2 - 2)


def _acc_rows(n):
    return -(-(n + 1) // 128) * 128


def _make_sc_degree(n, cpw):
    n16 = _acc_rows(n)
    rpt = n16 // _NS
    mesh = plsc.VectorSubcoreMesh(core_axis_name="c", subcore_axis_name="s")

    @functools.partial(
        pl.kernel,
        out_type=jax.ShapeDtypeStruct((_NC, n16, 16), jnp.float32),
        mesh=mesh,
        scratch_types=[
            pltpu.VMEM((_K,), jnp.int32),
            pltpu.VMEM((_K, 16), jnp.float32),
            pltpu.VMEM_SHARED((n16, 16), jnp.float32),
        ],
    )
    def deg_kernel(dst_hbm, z16_hbm, out_hbm, didx, ones_v, acc):
        cid = lax.axis_index("c")
        sid = lax.axis_index("s")
        wid = cid * _NS + sid
        ebase = wid * (cpw * _K)

        pltpu.sync_copy(z16_hbm.at[pl.ds(sid * rpt, rpt)],
                        acc.at[pl.ds(sid * rpt, rpt)])
        for r in range(_K):
            ones_v[r, :] = jnp.full((16,), 1.0, jnp.float32)
        plsc.subcore_barrier()

        def body(c, carry):
            pltpu.sync_copy(dst_hbm.at[pl.ds(ebase + c * _K, _K)], didx)
            pltpu.sync_copy(ones_v, acc.at[didx], add=True)
            return carry

        lax.fori_loop(0, cpw, body, 0)
        plsc.subcore_barrier()
        pltpu.sync_copy(acc.at[pl.ds(sid * rpt, rpt)],
                        out_hbm.at[cid, pl.ds(sid * rpt, rpt)])

    return deg_kernel


def _make_sc_scatter(n, cpw0, cpw1):
    n16 = _acc_rows(n)
    rpt = n16 // _NS
    mesh = plsc.VectorSubcoreMesh(core_axis_name="c", subcore_axis_name="s")

    @functools.partial(
        pl.kernel,
        out_type=jax.ShapeDtypeStruct((_NC, n16, _D), jnp.float32),
        mesh=mesh,
        scratch_types=[
            pltpu.VMEM((_K,), jnp.int32),
            pltpu.VMEM((_K,), jnp.int32),
            pltpu.VMEM((_K,), jnp.int32),
            pltpu.VMEM((_K,), jnp.int32),
            pltpu.VMEM((_K,), jnp.int32),
            pltpu.VMEM((_K,), jnp.int32),
            pltpu.VMEM((_K, _D), jnp.float32),
            pltpu.VMEM((_K, _D), jnp.float32),
            pltpu.VMEM((_K, _D), jnp.float32),
            pltpu.VMEM_SHARED((n16, _D), jnp.float32),
            pltpu.SemaphoreType.DMA,
            pltpu.SemaphoreType.DMA,
            pltpu.SemaphoreType.DMA,
            pltpu.SemaphoreType.DMA,
            pltpu.SemaphoreType.DMA,
            pltpu.SemaphoreType.DMA,
            pltpu.SemaphoreType.DMA,
            pltpu.SemaphoreType.DMA,
            pltpu.SemaphoreType.DMA,
            pltpu.SemaphoreType.DMA,
        ],
    )
    def sc_kernel(hp_hbm, src_hbm, dst_hbm, z_hbm, out_hbm,
                  sidx0, sidx1, sidx2, didx0, didx1, didx2,
                  rows0, rows1, rows2, acc,
                  ism0, ism1, ism2, idm0, idm1, idm2,
                  gsem0, gsem1, gsem2, ssem):
        cid = lax.axis_index("c")
        sid = lax.axis_index("s")
        my_cpw = jnp.where(cid == 0, cpw0, cpw1)
        base_chunk = jnp.where(cid == 0, sid * cpw0,
                               _NS * cpw0 + sid * cpw1)

        pltpu.sync_copy(z_hbm.at[pl.ds(sid * rpt, rpt)],
                        acc.at[pl.ds(sid * rpt, rpt)])
        plsc.subcore_barrier()

        sidx = (sidx0, sidx1, sidx2)
        didx = (didx0, didx1, didx2)
        rows = (rows0, rows1, rows2)
        isms = (ism0, ism1, ism2)
        idms = (idm0, idm1, idm2)
        gsms = (gsem0, gsem1, gsem2)

        def pair_body(c2):
            ebase = (base_chunk + 2 * c2) * _K
            fs0 = pltpu.async_copy(src_hbm.at[pl.ds(ebase, _K)],
                                   sidx[0], isms[0])
            fs1 = pltpu.async_copy(src_hbm.at[pl.ds(ebase + _K, _K)],
                                   sidx[1], isms[1])
            fd0 = pltpu.async_copy(dst_hbm.at[pl.ds(ebase, _K)],
                                   didx[0], idms[0])
            fd1 = pltpu.async_copy(dst_hbm.at[pl.ds(ebase + _K, _K)],
                                   didx[1], idms[1])
            fs0.wait()
            g0 = pltpu.async_copy(hp_hbm.at[sidx[0]], rows[0], gsms[0])
            fs1.wait()
            g1 = pltpu.async_copy(hp_hbm.at[sidx[1]], rows[1], gsms[1])
            fd0.wait()
            g0.wait()
            s0 = pltpu.async_copy(rows[0], acc.at[didx[0]], ssem, add=True)
            fd1.wait()
            g1.wait()
            s1 = pltpu.async_copy(rows[1], acc.at[didx[1]], ssem, add=True)
            s0.wait()
            s1.wait()

        def pair(c2, carry):
            @pl.when(2 * c2 < my_cpw)
            def _():
                pair_body(c2)
            return carry

        lax.fori_loop(0, max(cpw0, cpw1) // 2, pair, 0)

        plsc.subcore_barrier()
        pltpu.sync_copy(acc.at[pl.ds(sid * rpt, rpt)],
                        out_hbm.at[cid, pl.ds(sid * rpt, rpt)])

    return sc_kernel


def _dinv_from_partials(pd):
    deg = 1.0 + pd[0, :, 0] + pd[1, :, 0]
    return lax.rsqrt(deg)


def _tc_first_body(pd_ref, x_ref, w_ref, o_ref):
    dinv = _dinv_from_partials(pd_ref[...])
    h = jnp.dot(x_ref[...], w_ref[...], preferred_element_type=jnp.float32)
    o_ref[...] = h * dinv[:, None]


def _tc_mid_body(pd_ref, pm_ref, hp_ref, b_ref, w_ref, o_ref):
    dinv = _dinv_from_partials(pd_ref[...])
    pm = pm_ref[...]
    u = dinv[:, None] * (pm[0] + pm[1] + hp_ref[...]) + b_ref[...][None, :]
    a = jnp.maximum(u, 0.0)
    h = jnp.dot(a, w_ref[...], preferred_element_type=jnp.float32)
    o_ref[...] = h * dinv[:, None]


def _tc_final_body(pd_ref, pm_ref, hp_ref, b_ref, o_ref):
    dinv = _dinv_from_partials(pd_ref[...])
    pm = pm_ref[...]
    o_ref[...] = (dinv[:, None] * (pm[0] + pm[1] + hp_ref[...])
                  + b_ref[...][None, :])


def _row_block(n):
    for r in (400, 200, 1000, 8):
        if n % r == 0 and r % 8 == 0:
            return r
    return n


def _tc_first(pdeg, x, w):
    n = x.shape[0]
    r = _row_block(n)
    return pl.pallas_call(
        _tc_first_body,
        grid=(n // r,),
        in_specs=[
            pl.BlockSpec((2, r, 16), lambda i: (0, i, 0)),
            pl.BlockSpec((r, _D), lambda i: (i, 0)),
            pl.BlockSpec((_D, _D), lambda i: (0, 0)),
        ],
        out_specs=pl.BlockSpec((r, _D), lambda i: (i, 0)),
        out_shape=jax.ShapeDtypeStruct((n, _D), jnp.float32),
    )(pdeg, x, w)


def _tc_mid(pdeg, pm, hp, b, w):
    n = hp.shape[0]
    r = _row_block(n)
    return pl.pallas_call(
        _tc_mid_body,
        grid=(n // r,),
        in_specs=[
            pl.BlockSpec((2, r, 16), lambda i: (0, i, 0)),
            pl.BlockSpec((2, r, _D), lambda i: (0, i, 0)),
            pl.BlockSpec((r, _D), lambda i: (i, 0)),
            pl.BlockSpec((_D,), lambda i: (0,)),
            pl.BlockSpec((_D, _D), lambda i: (0, 0)),
        ],
        out_specs=pl.BlockSpec((r, _D), lambda i: (i, 0)),
        out_shape=jax.ShapeDtypeStruct((n, _D), jnp.float32),
    )(pdeg, pm, hp, b, w)


def _tc_final(pdeg, pm, hp, b):
    n = hp.shape[0]
    r = _row_block(n)
    return pl.pallas_call(
        _tc_final_body,
        grid=(n // r,),
        in_specs=[
            pl.BlockSpec((2, r, 16), lambda i: (0, i, 0)),
            pl.BlockSpec((2, r, _D), lambda i: (0, i, 0)),
            pl.BlockSpec((r, _D), lambda i: (i, 0)),
            pl.BlockSpec((_D,), lambda i: (0,)),
        ],
        out_specs=pl.BlockSpec((r, _D), lambda i: (i, 0)),
        out_shape=jax.ShapeDtypeStruct((n, _D), jnp.float32),
    )(pdeg, pm, hp, b)


def kernel(x, edge_index, W1, b1, W2, b2, W3, b3):
    n, d = x.shape
    e = edge_index.shape[1]
    need = -(-e // (_NS * _K))
    cpw0 = max(2, int(round(need * 0.75 / 2)) * 2)
    cpw1 = max(2, -(-(need - cpw0) // 2) * 2)
    e_pad = _NS * (cpw0 + cpw1) * _K
    cpw = (cpw0 + cpw1) // 2
    n16 = _acc_rows(n)
    pad = e_pad - e
    src_p = jnp.concatenate(
        [edge_index[0], jnp.zeros((pad,), edge_index.dtype)])
    dst_p = jnp.concatenate(
        [edge_index[1], jnp.full((pad,), n, edge_index.dtype)])
    z = jnp.zeros((n16, d), jnp.float32)
    z16 = jnp.zeros((n16, 16), jnp.float32)

    sc_deg = _make_sc_degree(n, cpw)
    sc_msgs = _make_sc_scatter(n, cpw0, cpw1)

    pdeg = sc_deg(dst_p, z16)
    hp = _tc_first(pdeg, x, W1)
    pm = sc_msgs(hp, src_p, dst_p, z)
    hp = _tc_mid(pdeg, pm, hp, b1, W2)
    pm = sc_msgs(hp, src_p, dst_p, z)
    hp = _tc_mid(pdeg, pm, hp, b2, W3)
    pm = sc_msgs(hp, src_p, dst_p, z)
    return _tc_final(pdeg, pm, hp, b3)


# speedup vs baseline: 1.0193x; 1.0193x over previous
import functools

import jax
import jax.numpy as jnp
from jax import lax
from jax.experimental import pallas as pl
from jax.experimental.pallas import tpu as pltpu
from jax.experimental.pallas import tpu_sc as plsc

_NC = 2
_NS = 16
_NW = _NC * _NS
_K = 128
_D = 128


def _SPLIT0(total2):
    c0 = int(round(total2 * 0.90 / 2)) * 2
    return min(max(c0, 2), total2 - 2)


def _acc_rows(n):
    return -(-(n + 1) // 128) * 128


def _make_sc_degree(n, cpw):
    n16 = _acc_rows(n)
    rpt = n16 // _NS
    mesh = plsc.VectorSubcoreMesh(core_axis_name="c", subcore_axis_name="s")

    @functools.partial(
        pl.kernel,
        out_type=jax.ShapeDtypeStruct((_NC, n16, 16), jnp.float32),
        mesh=mesh,
        scratch_types=[
            pltpu.VMEM((_K,), jnp.int32),
            pltpu.VMEM((_K, 16), jnp.float32),
            pltpu.VMEM_SHARED((n16, 16), jnp.float32),
        ],
    )
    def deg_kernel(dst_hbm, z16_hbm, out_hbm, didx, ones_v, acc):
        cid = lax.axis_index("c")
        sid = lax.axis_index("s")
        wid = cid * _NS + sid
        ebase = wid * (cpw * _K)

        pltpu.sync_copy(z16_hbm.at[pl.ds(sid * rpt, rpt)],
                        acc.at[pl.ds(sid * rpt, rpt)])
        for r in range(_K):
            ones_v[r, :] = jnp.full((16,), 1.0, jnp.float32)
        plsc.subcore_barrier()

        def body(c, carry):
            pltpu.sync_copy(dst_hbm.at[pl.ds(ebase + c * _K, _K)], didx)
            pltpu.sync_copy(ones_v, acc.at[didx], add=True)
            return carry

        lax.fori_loop(0, cpw, body, 0)
        plsc.subcore_barrier()
        pltpu.sync_copy(acc.at[pl.ds(sid * rpt, rpt)],
                        out_hbm.at[cid, pl.ds(sid * rpt, rpt)])

    return deg_kernel


def _make_sc_scatter(n, cpw0, cpw1):
    n16 = _acc_rows(n)
    rpt = n16 // _NS
    mesh = plsc.VectorSubcoreMesh(core_axis_name="c", subcore_axis_name="s")

    @functools.partial(
        pl.kernel,
        out_type=jax.ShapeDtypeStruct((_NC, n16, _D), jnp.float32),
        mesh=mesh,
        scratch_types=[
            pltpu.VMEM((_K,), jnp.int32),
            pltpu.VMEM((_K,), jnp.int32),
            pltpu.VMEM((_K,), jnp.int32),
            pltpu.VMEM((_K,), jnp.int32),
            pltpu.VMEM((_K,), jnp.int32),
            pltpu.VMEM((_K,), jnp.int32),
            pltpu.VMEM((_K, _D), jnp.float32),
            pltpu.VMEM((_K, _D), jnp.float32),
            pltpu.VMEM((_K, _D), jnp.float32),
            pltpu.VMEM_SHARED((n16, _D), jnp.float32),
            pltpu.SemaphoreType.DMA,
            pltpu.SemaphoreType.DMA,
            pltpu.SemaphoreType.DMA,
            pltpu.SemaphoreType.DMA,
            pltpu.SemaphoreType.DMA,
            pltpu.SemaphoreType.DMA,
            pltpu.SemaphoreType.DMA,
            pltpu.SemaphoreType.DMA,
            pltpu.SemaphoreType.DMA,
            pltpu.SemaphoreType.DMA,
        ],
    )
    def sc_kernel(hp_hbm, src_hbm, dst_hbm, z_hbm, out_hbm,
                  sidx0, sidx1, sidx2, didx0, didx1, didx2,
                  rows0, rows1, rows2, acc,
                  ism0, ism1, ism2, idm0, idm1, idm2,
                  gsem0, gsem1, gsem2, ssem):
        cid = lax.axis_index("c")
        sid = lax.axis_index("s")
        my_cpw = jnp.where(cid == 0, cpw0, cpw1)
        base_chunk = jnp.where(cid == 0, sid * cpw0,
                               _NS * cpw0 + sid * cpw1)

        pltpu.sync_copy(z_hbm.at[pl.ds(sid * rpt, rpt)],
                        acc.at[pl.ds(sid * rpt, rpt)])
        plsc.subcore_barrier()

        sidx = (sidx0, sidx1, sidx2)
        didx = (didx0, didx1, didx2)
        rows = (rows0, rows1, rows2)
        isms = (ism0, ism1, ism2)
        idms = (idm0, idm1, idm2)
        gsms = (gsem0, gsem1, gsem2)

        def pair_body(c2):
            ebase = (base_chunk + 2 * c2) * _K
            fs0 = pltpu.async_copy(src_hbm.at[pl.ds(ebase, _K)],
                                   sidx[0], isms[0])
            fs1 = pltpu.async_copy(src_hbm.at[pl.ds(ebase + _K, _K)],
                                   sidx[1], isms[1])
            fd0 = pltpu.async_copy(dst_hbm.at[pl.ds(ebase, _K)],
                                   didx[0], idms[0])
            fd1 = pltpu.async_copy(dst_hbm.at[pl.ds(ebase + _K, _K)],
                                   didx[1], idms[1])
            fs0.wait()
            g0 = pltpu.async_copy(hp_hbm.at[sidx[0]], rows[0], gsms[0])
            fs1.wait()
            g1 = pltpu.async_copy(hp_hbm.at[sidx[1]], rows[1], gsms[1])
            fd0.wait()
            g0.wait()
            s0 = pltpu.async_copy(rows[0], acc.at[didx[0]], ssem, add=True)
            fd1.wait()
            g1.wait()
            s1 = pltpu.async_copy(rows[1], acc.at[didx[1]], ssem, add=True)
            s0.wait()
            s1.wait()

        def pair(c2, carry):
            @pl.when(2 * c2 < my_cpw)
            def _():
                pair_body(c2)
            return carry

        lax.fori_loop(0, max(cpw0, cpw1) // 2, pair, 0)

        plsc.subcore_barrier()
        pltpu.sync_copy(acc.at[pl.ds(sid * rpt, rpt)],
                        out_hbm.at[cid, pl.ds(sid * rpt, rpt)])

    return sc_kernel


def _dinv_from_partials(pd):
    deg = 1.0 + pd[0, :, 0] + pd[1, :, 0]
    return lax.rsqrt(deg)


def _tc_first_body(pd_ref, x_ref, w_ref, o_ref):
    dinv = _dinv_from_partials(pd_ref[...])
    h = jnp.dot(x_ref[...], w_ref[...], preferred_element_type=jnp.float32)
    o_ref[...] = h * dinv[:, None]


def _tc_mid_body(pd_ref, pm_ref, hp_ref, b_ref, w_ref, o_ref):
    dinv = _dinv_from_partials(pd_ref[...])
    pm = pm_ref[...]
    u = dinv[:, None] * (pm[0] + pm[1] + hp_ref[...]) + b_ref[...][None, :]
    a = jnp.maximum(u, 0.0)
    h = jnp.dot(a, w_ref[...], preferred_element_type=jnp.float32)
    o_ref[...] = h * dinv[:, None]


def _tc_final_body(pd_ref, pm_ref, hp_ref, b_ref, o_ref):
    dinv = _dinv_from_partials(pd_ref[...])
    pm = pm_ref[...]
    o_ref[...] = (dinv[:, None] * (pm[0] + pm[1] + hp_ref[...])
                  + b_ref[...][None, :])


def _row_block(n):
    for r in (400, 200, 1000, 8):
        if n % r == 0 and r % 8 == 0:
            return r
    return n


def _tc_first(pdeg, x, w):
    n = x.shape[0]
    r = _row_block(n)
    return pl.pallas_call(
        _tc_first_body,
        grid=(n // r,),
        in_specs=[
            pl.BlockSpec((2, r, 16), lambda i: (0, i, 0)),
            pl.BlockSpec((r, _D), lambda i: (i, 0)),
            pl.BlockSpec((_D, _D), lambda i: (0, 0)),
        ],
        out_specs=pl.BlockSpec((r, _D), lambda i: (i, 0)),
        out_shape=jax.ShapeDtypeStruct((n, _D), jnp.float32),
    )(pdeg, x, w)


def _tc_mid(pdeg, pm, hp, b, w):
    n = hp.shape[0]
    r = _row_block(n)
    return pl.pallas_call(
        _tc_mid_body,
        grid=(n // r,),
        in_specs=[
            pl.BlockSpec((2, r, 16), lambda i: (0, i, 0)),
            pl.BlockSpec((2, r, _D), lambda i: (0, i, 0)),
            pl.BlockSpec((r, _D), lambda i: (i, 0)),
            pl.BlockSpec((_D,), lambda i: (0,)),
            pl.BlockSpec((_D, _D), lambda i: (0, 0)),
        ],
        out_specs=pl.BlockSpec((r, _D), lambda i: (i, 0)),
        out_shape=jax.ShapeDtypeStruct((n, _D), jnp.float32),
    )(pdeg, pm, hp, b, w)


def _tc_final(pdeg, pm, hp, b):
    n = hp.shape[0]
    r = _row_block(n)
    return pl.pallas_call(
        _tc_final_body,
        grid=(n // r,),
        in_specs=[
            pl.BlockSpec((2, r, 16), lambda i: (0, i, 0)),
            pl.BlockSpec((2, r, _D), lambda i: (0, i, 0)),
            pl.BlockSpec((r, _D), lambda i: (i, 0)),
            pl.BlockSpec((_D,), lambda i: (0,)),
        ],
        out_specs=pl.BlockSpec((r, _D), lambda i: (i, 0)),
        out_shape=jax.ShapeDtypeStruct((n, _D), jnp.float32),
    )(pdeg, pm, hp, b)


def kernel(x, edge_index, W1, b1, W2, b2, W3, b3):
    n, d = x.shape
    e = edge_index.shape[1]
    need = -(-e // (_NS * _K))
    cpw0 = max(2, int(round(need * 0.78 / 2)) * 2)
    cpw1 = max(2, -(-(need - cpw0) // 2) * 2)
    e_pad = _NS * (cpw0 + cpw1) * _K
    cpw = (cpw0 + cpw1) // 2
    n16 = _acc_rows(n)
    pad = e_pad - e
    src_p = jnp.concatenate(
        [edge_index[0], jnp.zeros((pad,), edge_index.dtype)])
    dst_p = jnp.concatenate(
        [edge_index[1], jnp.full((pad,), n, edge_index.dtype)])
    z = jnp.zeros((n16, d), jnp.float32)
    z16 = jnp.zeros((n16, 16), jnp.float32)

    sc_deg = _make_sc_degree(n, cpw)
    sc_msgs = _make_sc_scatter(n, cpw0, cpw1)

    pdeg = sc_deg(dst_p, z16)
    hp = _tc_first(pdeg, x, W1)
    pm = sc_msgs(hp, src_p, dst_p, z)
    hp = _tc_mid(pdeg, pm, hp, b1, W2)
    pm = sc_msgs(hp, src_p, dst_p, z)
    hp = _tc_mid(pdeg, pm, hp, b2, W3)
    pm = sc_msgs(hp, src_p, dst_p, z)
    return _tc_final(pdeg, pm, hp, b3)
